# baseline (speedup 1.0000x reference)
import functools
import math

import jax
import jax.numpy as jnp
from jax import lax
from jax.experimental import pallas as pl
from jax.experimental.pallas import tpu as pltpu
from jax.experimental.pallas import tpu_sc as plsc

_NUM_HEADS = 32
_NUM_SEGMENTS = 32
_NUM_BUCKETS = 512
_MAX_DISTANCE = 2048

_NC = 2
_NS = 16
_NW = _NC * _NS
_LANES = 16
_HALF = 1024


def _abs_bucket(rp):
    half = _NUM_BUCKETS // 2
    rb = (rp > 0).astype(jnp.int32) * half
    x = jnp.abs(rp)
    max_exact = half // 2
    is_small = x < max_exact
    rp_f = jnp.maximum(x.astype(jnp.float32), 1.0)
    large = max_exact + (
        jnp.log(rp_f / max_exact)
        / math.log(_MAX_DISTANCE / max_exact)
        * (half - max_exact)
    ).astype(jnp.int32)
    large = jnp.minimum(large, half - 1)
    return rb + jnp.where(is_small, x, large)


def _prep_kernel(a_cols):
    def _k(rel_bias_ref, relt_ref, ftab_ref):
        relt_ref[...] = jnp.transpose(rel_bias_ref[...])
        d = lax.broadcasted_iota(jnp.int32, (1, a_cols), 1)
        ftab_ref[...] = _abs_bucket(d - (a_cols // 2 - 1))

    return _k


def _tc_prep(rel_bias, q_len, k_len):
    table_rows, nh = rel_bias.shape
    a_cols = q_len + k_len
    return pl.pallas_call(
        _prep_kernel(a_cols),
        out_shape=(
            jax.ShapeDtypeStruct((nh, table_rows), jnp.float32),
            jax.ShapeDtypeStruct((1, a_cols), jnp.int32),
        ),
    )(rel_bias)


def _sc_kernel(q_len, k_len, table_rows):
    rows_per_w = q_len // _NW
    n_half = k_len // _HALF
    n_chunk = _HALF // _LANES
    relt_n = _NUM_HEADS * table_rows
    a_cols = q_len + k_len
    mesh = plsc.VectorSubcoreMesh(core_axis_name="c", subcore_axis_name="s")

    @functools.partial(
        pl.kernel,
        mesh=mesh,
        out_type=jax.ShapeDtypeStruct((1, _NUM_HEADS, q_len, k_len), jnp.float32),
        scratch_types=[
            pltpu.VMEM((relt_n,), jnp.float32),
            pltpu.VMEM((a_cols,), jnp.int32),
            pltpu.VMEM((k_len,), jnp.int32),
            pltpu.VMEM((q_len,), jnp.int32),
            pltpu.VMEM((2, _NUM_HEADS, _HALF), jnp.float32),
            pltpu.SemaphoreType.DMA,
            pltpu.SemaphoreType.DMA,
        ],
        compiler_params=pltpu.CompilerParams(needs_layout_passes=False),
    )
    def _k(relt_hbm, ftab_hbm, ks_hbm, qseg_hbm, out_hbm,
           relt_v, ftab_v, ks_v, qseg_v, obuf, sem0, sem1):
        wid = lax.axis_index("s") * _NC + lax.axis_index("c")
        base_q = wid * rows_per_w
        sems = [sem0, sem1]

        pltpu.sync_copy(relt_hbm, relt_v)
        pltpu.sync_copy(ftab_hbm, ftab_v)
        pltpu.sync_copy(ks_hbm, ks_v)
        pltpu.sync_copy(qseg_hbm, qseg_v)

        def _copy(slot, q, half):
            return pltpu.make_async_copy(
                obuf.at[slot],
                out_hbm.at[0, :, q, pl.ds(half * _HALF, _HALF)],
                sems[slot],
            )

        def row_body(r, carry):
            q = base_q + r
            qvv = plsc.load_gather(
                qseg_v, [jnp.zeros((_LANES,), jnp.int32) + q]
            )
            for half in range(n_half):
                slot = half

                @pl.when(r > 0)
                def _wait_prev():
                    _copy(slot, q - 1, half).wait()

                def chunk_body(c, carry2):
                    off = half * _HALF + c * _LANES
                    ksv = ks_v[pl.ds(off, _LANES)]
                    absb = ftab_v[pl.ds((q_len - 1) - q + off, _LANES)]
                    segb = _NUM_BUCKETS + qvv * _NUM_SEGMENTS + ksv
                    bucket = jnp.where(ksv == qvv, absb, segb)
                    for h in range(_NUM_HEADS):
                        val = plsc.load_gather(relt_v, [bucket + h * table_rows])
                        obuf[slot, h, pl.ds(c * _LANES, _LANES)] = val
                    return carry2

                lax.fori_loop(0, n_chunk, chunk_body, 0)
                _copy(slot, q, half).start()
            return carry

        lax.fori_loop(0, rows_per_w, row_body, 0)
        for slot in range(n_half):
            _copy(slot, base_q + rows_per_w - 1, slot).wait()

    return _k


def kernel(key_pos, query_pos, key_segment, query_segment, rel_bias):
    del key_pos, query_pos
    k_len = key_segment.shape[1]
    q_len = query_segment.shape[1]
    table_rows = rel_bias.shape[0]
    relt, ftab = _tc_prep(rel_bias, q_len, k_len)
    out = _sc_kernel(q_len, k_len, table_rows)(
        relt.reshape(-1),
        ftab.reshape(-1),
        key_segment.reshape(-1),
        query_segment.reshape(-1),
    )
    return out


# speedup vs baseline: 3.3397x; 3.3397x over previous
import functools
import math

import jax
import jax.numpy as jnp
from jax import lax
from jax.experimental import pallas as pl
from jax.experimental.pallas import tpu as pltpu
from jax.experimental.pallas import tpu_sc as plsc

_NUM_HEADS = 32
_NUM_SEGMENTS = 32
_NUM_BUCKETS = 512
_MAX_DISTANCE = 2048

_QT = 32
_LANES = 16


def _abs_bucket(rp):
    half = _NUM_BUCKETS // 2
    rb = (rp > 0).astype(jnp.int32) * half
    x = jnp.abs(rp)
    max_exact = half // 2
    is_small = x < max_exact
    rp_f = jnp.maximum(x.astype(jnp.float32), 1.0)
    large = max_exact + (
        jnp.log(rp_f / max_exact)
        / math.log(_MAX_DISTANCE / max_exact)
        * (half - max_exact)
    ).astype(jnp.int32)
    large = jnp.minimum(large, half - 1)
    return rb + jnp.where(is_small, x, large)


def _ftab_prep(a_cols):
    def _k(ftab_ref):
        d = lax.broadcasted_iota(jnp.int32, (1, a_cols), 1)
        ftab_ref[...] = _abs_bucket(d - (a_cols // 2 - 1))

    return pl.pallas_call(
        _k, out_shape=jax.ShapeDtypeStruct((1, a_cols), jnp.int32)
    )()


def _sc_build_a(rel_bias_flat, ftab_flat, a_cols, table_rows):
    n_chunk = a_cols // _LANES
    relb_n = table_rows * _NUM_HEADS
    mesh = plsc.VectorSubcoreMesh(core_axis_name="c", subcore_axis_name="s")

    @functools.partial(
        pl.kernel,
        mesh=mesh,
        out_type=jax.ShapeDtypeStruct((_NUM_HEADS, a_cols), jnp.float32),
        scratch_types=[
            pltpu.VMEM((relb_n,), jnp.float32),
            pltpu.VMEM((a_cols,), jnp.int32),
            pltpu.VMEM((a_cols,), jnp.float32),
        ],
        compiler_params=pltpu.CompilerParams(needs_layout_passes=False),
    )
    def _k(relb_hbm, ftab_hbm, a_hbm, relb_v, ftab_v, abuf):
        h = lax.axis_index("s") * 2 + lax.axis_index("c")

        pltpu.sync_copy(relb_hbm, relb_v)
        pltpu.sync_copy(ftab_hbm, ftab_v)

        def chunk_body(c, carry):
            idx = ftab_v[pl.ds(c * _LANES, _LANES)] * _NUM_HEADS + h
            abuf[pl.ds(c * _LANES, _LANES)] = plsc.load_gather(relb_v, [idx])
            return carry

        lax.fori_loop(0, n_chunk, chunk_body, 0)
        pltpu.sync_copy(abuf, a_hbm.at[h])

    return _k(rel_bias_flat, ftab_flat)


def _make_tc_kernel(q_len, k_len, table_rows, a_cols):
    def _kernel(
        qseg_ref,
        rel_bias_ref,
        ks_ref,
        a_in_ref,
        out_ref,
        a_scr,
        ablk_scr,
        o32_scr,
        ksb_scr,
    ):
        i = pl.program_id(0)

        @pl.when(i == 0)
        def _init():
            a_scr[...] = a_in_ref[...].astype(jnp.bfloat16)
            o32_scr[...] = (
                lax.broadcasted_iota(jnp.int32, (_NUM_SEGMENTS, k_len), 0)
                == ks_ref[0:1, :]
            ).astype(jnp.bfloat16)
            ksb_scr[...] = jnp.broadcast_to(ks_ref[0:1, :], (_NUM_HEADS, k_len))

        q0 = i * _QT

        base = (q_len - 1) - q0 - (_QT - 1)
        ablk_scr[...] = pltpu.roll(a_scr[...], -base, axis=1)

        ksb = ksb_scr[...]
        o32 = o32_scr[...]
        for d_ in range(_QT):
            qv = qseg_ref[q0 + d_]
            s_d = rel_bias_ref[
                pl.ds(_NUM_BUCKETS + qv * _NUM_SEGMENTS, _NUM_SEGMENTS), :
            ]
            g_d = jnp.transpose(s_d).astype(jnp.bfloat16)
            seg_row = jnp.dot(g_d, o32, preferred_element_type=jnp.float32)
            off = _QT - 1 - d_
            a_row = ablk_scr[:, off : off + k_len].astype(jnp.float32)
            same = ksb == qv
            out_ref[0, :, d_, :] = jnp.where(same, a_row, seg_row)

    return _kernel


def _tc_main(query_segment_1d, key_segment_2d, rel_bias, a_table, q_len, k_len):
    table_rows = rel_bias.shape[0]
    a_cols = a_table.shape[1]
    grid = q_len // _QT
    return pl.pallas_call(
        _make_tc_kernel(q_len, k_len, table_rows, a_cols),
        grid_spec=pltpu.PrefetchScalarGridSpec(
            num_scalar_prefetch=1,
            grid=(grid,),
            in_specs=[
                pl.BlockSpec((table_rows, _NUM_HEADS), lambda i, s: (0, 0)),
                pl.BlockSpec((1, k_len), lambda i, s: (0, 0)),
                pl.BlockSpec((_NUM_HEADS, a_cols), lambda i, s: (0, 0)),
            ],
            out_specs=pl.BlockSpec(
                (1, _NUM_HEADS, _QT, k_len), lambda i, s: (0, 0, i, 0)
            ),
            scratch_shapes=[
                pltpu.VMEM((_NUM_HEADS, a_cols), jnp.bfloat16),
                pltpu.VMEM((_NUM_HEADS, a_cols), jnp.bfloat16),
                pltpu.VMEM((_NUM_SEGMENTS, k_len), jnp.bfloat16),
                pltpu.VMEM((_NUM_HEADS, k_len), jnp.int32),
            ],
        ),
        out_shape=jax.ShapeDtypeStruct((1, _NUM_HEADS, q_len, k_len), jnp.float32),
        compiler_params=pltpu.CompilerParams(
            dimension_semantics=("arbitrary",),
        ),
    )(query_segment_1d, rel_bias, key_segment_2d, a_table)


def kernel(key_pos, query_pos, key_segment, query_segment, rel_bias):
    del key_pos, query_pos
    k_len = key_segment.shape[1]
    q_len = query_segment.shape[1]
    table_rows = rel_bias.shape[0]
    a_cols = q_len + k_len
    ftab = _ftab_prep(a_cols)
    a_table = _sc_build_a(
        rel_bias.reshape(-1), ftab.reshape(-1), a_cols, table_rows
    )
    return _tc_main(
        query_segment.reshape(-1),
        key_segment.reshape(1, k_len),
        rel_bias,
        a_table,
        q_len,
        k_len,
    )


# speedup vs baseline: 3.3890x; 1.0148x over previous
import functools
import math

import jax
import jax.numpy as jnp
from jax import lax
from jax.experimental import pallas as pl
from jax.experimental.pallas import tpu as pltpu
from jax.experimental.pallas import tpu_sc as plsc

_NUM_HEADS = 32
_NUM_SEGMENTS = 32
_NUM_BUCKETS = 512
_MAX_DISTANCE = 2048

_QT = 64
_LANES = 16


def _abs_bucket(rp):
    half = _NUM_BUCKETS // 2
    rb = (rp > 0).astype(jnp.int32) * half
    x = jnp.abs(rp)
    max_exact = half // 2
    is_small = x < max_exact
    rp_f = jnp.maximum(x.astype(jnp.float32), 1.0)
    large = max_exact + (
        jnp.log(rp_f / max_exact)
        / math.log(_MAX_DISTANCE / max_exact)
        * (half - max_exact)
    ).astype(jnp.int32)
    large = jnp.minimum(large, half - 1)
    return rb + jnp.where(is_small, x, large)


def _ftab_prep(a_cols):
    def _k(ftab_ref):
        d = lax.broadcasted_iota(jnp.int32, (1, a_cols), 1)
        ftab_ref[...] = _abs_bucket(d - (a_cols // 2 - 1))

    return pl.pallas_call(
        _k, out_shape=jax.ShapeDtypeStruct((1, a_cols), jnp.int32)
    )()


def _sc_build_a(rel_bias_flat, ftab_flat, a_cols, table_rows):
    n_chunk = a_cols // _LANES
    relb_n = table_rows * _NUM_HEADS
    mesh = plsc.VectorSubcoreMesh(core_axis_name="c", subcore_axis_name="s")

    @functools.partial(
        pl.kernel,
        mesh=mesh,
        out_type=jax.ShapeDtypeStruct((_NUM_HEADS, a_cols), jnp.float32),
        scratch_types=[
            pltpu.VMEM((relb_n,), jnp.float32),
            pltpu.VMEM((a_cols,), jnp.int32),
            pltpu.VMEM((a_cols,), jnp.float32),
        ],
        compiler_params=pltpu.CompilerParams(needs_layout_passes=False),
    )
    def _k(relb_hbm, ftab_hbm, a_hbm, relb_v, ftab_v, abuf):
        h = lax.axis_index("s") * 2 + lax.axis_index("c")

        pltpu.sync_copy(relb_hbm, relb_v)
        pltpu.sync_copy(ftab_hbm, ftab_v)

        def chunk_body(c, carry):
            idx = ftab_v[pl.ds(c * _LANES, _LANES)] * _NUM_HEADS + h
            abuf[pl.ds(c * _LANES, _LANES)] = plsc.load_gather(relb_v, [idx])
            return carry

        lax.fori_loop(0, n_chunk, chunk_body, 0)
        pltpu.sync_copy(abuf, a_hbm.at[h])

    return _k(rel_bias_flat, ftab_flat)


def _make_tc_kernel(q_len, k_len, table_rows, a_cols):
    def _kernel(
        qseg_ref,
        rel_bias_ref,
        ks_ref,
        a_in_ref,
        out_ref,
        a_scr,
        ablk_scr,
        o32_scr,
        ksb_scr,
    ):
        i = pl.program_id(0)

        @pl.when(i == 0)
        def _init():
            a_scr[...] = a_in_ref[...].astype(jnp.bfloat16)
            o32_scr[...] = (
                lax.broadcasted_iota(jnp.int32, (_NUM_SEGMENTS, k_len), 0)
                == ks_ref[0:1, :]
            ).astype(jnp.bfloat16)
            ksb_scr[...] = jnp.broadcast_to(ks_ref[0:1, :], (_NUM_HEADS, k_len))

        q0 = i * _QT

        base = (q_len - 1) - q0 - (_QT - 1)
        ablk_scr[...] = pltpu.roll(a_scr[...], -base, axis=1)

        ksb = ksb_scr[...]
        o32 = o32_scr[...]
        for d_ in range(_QT):
            qv = qseg_ref[q0 + d_]
            s_d = rel_bias_ref[
                pl.ds(_NUM_BUCKETS + qv * _NUM_SEGMENTS, _NUM_SEGMENTS), :
            ]
            g_d = jnp.transpose(s_d).astype(jnp.bfloat16)
            seg_row = jnp.dot(g_d, o32, preferred_element_type=jnp.float32)
            off = _QT - 1 - d_
            a_row = ablk_scr[:, off : off + k_len].astype(jnp.float32)
            same = ksb == qv
            out_ref[0, :, d_, :] = jnp.where(same, a_row, seg_row)

    return _kernel


def _tc_main(query_segment_1d, key_segment_2d, rel_bias, a_table, q_len, k_len):
    table_rows = rel_bias.shape[0]
    a_cols = a_table.shape[1]
    grid = q_len // _QT
    return pl.pallas_call(
        _make_tc_kernel(q_len, k_len, table_rows, a_cols),
        grid_spec=pltpu.PrefetchScalarGridSpec(
            num_scalar_prefetch=1,
            grid=(grid,),
            in_specs=[
                pl.BlockSpec((table_rows, _NUM_HEADS), lambda i, s: (0, 0)),
                pl.BlockSpec((1, k_len), lambda i, s: (0, 0)),
                pl.BlockSpec((_NUM_HEADS, a_cols), lambda i, s: (0, 0)),
            ],
            out_specs=pl.BlockSpec(
                (1, _NUM_HEADS, _QT, k_len), lambda i, s: (0, 0, i, 0)
            ),
            scratch_shapes=[
                pltpu.VMEM((_NUM_HEADS, a_cols), jnp.bfloat16),
                pltpu.VMEM((_NUM_HEADS, a_cols), jnp.bfloat16),
                pltpu.VMEM((_NUM_SEGMENTS, k_len), jnp.bfloat16),
                pltpu.VMEM((_NUM_HEADS, k_len), jnp.int32),
            ],
        ),
        out_shape=jax.ShapeDtypeStruct((1, _NUM_HEADS, q_len, k_len), jnp.float32),
        compiler_params=pltpu.CompilerParams(
            dimension_semantics=("arbitrary",),
        ),
    )(query_segment_1d, rel_bias, key_segment_2d, a_table)


def kernel(key_pos, query_pos, key_segment, query_segment, rel_bias):
    del key_pos, query_pos
    k_len = key_segment.shape[1]
    q_len = query_segment.shape[1]
    table_rows = rel_bias.shape[0]
    a_cols = q_len + k_len
    ftab = _ftab_prep(a_cols)
    a_table = _sc_build_a(
        rel_bias.reshape(-1), ftab.reshape(-1), a_cols, table_rows
    )
    return _tc_main(
        query_segment.reshape(-1),
        key_segment.reshape(1, k_len),
        rel_bias,
        a_table,
        q_len,
        k_len,
    )
